# Initial kernel scaffold; baseline (speedup 1.0000x reference)
import functools

import jax
import jax.numpy as jnp
from jax import lax
from jax.experimental import pallas as pl
from jax.experimental.pallas import tpu as pltpu
from jax.experimental.pallas import tpu_sc as plsc

NUM_TEXT = 100000
NUM_POS = 2048
DIM = 64
B = 4096
L = 200

_NC = 2
_NS = 16
_NW = _NC * _NS
_ROWS_PER_W = B // _NW

_CHUNK0 = 128
_CHUNK1 = L - _CHUNK0


def _body(x_hbm, text_hbm, pos_hbm, out_hbm, idx_v, rows_v, pos_v, sem):
    wid = lax.axis_index("s") * _NC + lax.axis_index("c")

    pltpu.sync_copy(pos_hbm.at[pl.ds(0, L)], pos_v)

    def per_row(b_local, carry):
        b = wid * _ROWS_PER_W + b_local
        pltpu.sync_copy(x_hbm.at[b], idx_v)
        c0 = pltpu.async_copy(
            text_hbm.at[idx_v.at[pl.ds(0, _CHUNK0)]],
            rows_v.at[pl.ds(0, _CHUNK0)],
            sem,
        )
        c1 = pltpu.async_copy(
            text_hbm.at[idx_v.at[pl.ds(_CHUNK0, _CHUNK1)]],
            rows_v.at[pl.ds(_CHUNK0, _CHUNK1)],
            sem,
        )
        c0.wait()
        c1.wait()

        def add_row(l, carry2):
            for g in range(DIM // 16):
                sl = pl.ds(g * 16, 16)
                rows_v[l, sl] = rows_v[l, sl] + pos_v[l, sl]
            return carry2

        lax.fori_loop(0, L, add_row, 0)

        pltpu.sync_copy(rows_v, out_hbm.at[b])
        return carry

    lax.fori_loop(0, _ROWS_PER_W, per_row, 0)


@jax.jit
def kernel(x, text_table, pos_table):
    mesh = plsc.VectorSubcoreMesh(core_axis_name="c", subcore_axis_name="s")
    run = functools.partial(
        pl.kernel,
        out_type=jax.ShapeDtypeStruct((B, L, DIM), jnp.float32),
        mesh=mesh,
        scratch_types=[
            pltpu.VMEM((L,), jnp.int32),
            pltpu.VMEM((L, DIM), jnp.float32),
            pltpu.VMEM((L, DIM), jnp.float32),
            pltpu.SemaphoreType.DMA,
        ],
    )(_body)
    return run(x.astype(jnp.int32), text_table, pos_table)


# speedup vs baseline: 3.1155x; 3.1155x over previous
import functools

import jax
import jax.numpy as jnp
from jax import lax
from jax.experimental import pallas as pl
from jax.experimental.pallas import tpu as pltpu
from jax.experimental.pallas import tpu_sc as plsc

NUM_TEXT = 100000
NUM_POS = 2048
DIM = 64
B = 4096
L = 200

_NC = 2
_NS = 16
_NW = _NC * _NS
_ROWS_PER_W = B // _NW

_CHUNK0 = 128
_CHUNK1 = L - _CHUNK0


def _body(x_hbm, text_hbm, pos_hbm, out_hbm, idx_v, rows_v, pos_v, sem):
    wid = lax.axis_index("s") * _NC + lax.axis_index("c")

    pltpu.sync_copy(pos_hbm.at[pl.ds(0, L)], pos_v)

    def per_row(b_local, carry):
        b = wid * _ROWS_PER_W + b_local
        pltpu.sync_copy(x_hbm.at[b], idx_v)
        c0 = pltpu.async_copy(
            text_hbm.at[idx_v.at[pl.ds(0, _CHUNK0)]],
            rows_v.at[pl.ds(0, _CHUNK0)],
            sem,
        )
        c1 = pltpu.async_copy(
            text_hbm.at[idx_v.at[pl.ds(_CHUNK0, _CHUNK1)]],
            rows_v.at[pl.ds(_CHUNK0, _CHUNK1)],
            sem,
        )
        c0.wait()
        c1.wait()

        def add_row(l, carry2):
            for g in range(DIM // 16):
                sl = pl.ds(g * 16, 16)
                rows_v[l, sl] = rows_v[l, sl] + pos_v[l, sl]
            return carry2

        lax.fori_loop(0, L, add_row, 0)

        pltpu.sync_copy(rows_v, out_hbm.at[b])
        return carry

    lax.fori_loop(0, _ROWS_PER_W, per_row, 0)


@jax.jit
def kernel(x, text_table, pos_table):
    mesh = plsc.VectorSubcoreMesh(core_axis_name="c", subcore_axis_name="s")
    run = functools.partial(
        pl.kernel,
        out_type=jax.ShapeDtypeStruct((B, L, DIM), jnp.float32),
        mesh=mesh,
        scratch_types=[
            pltpu.VMEM((L,), jnp.int32),
            pltpu.VMEM((L, DIM), jnp.float32),
            pltpu.VMEM((L, DIM), jnp.float32),
            pltpu.SemaphoreType.DMA,
        ],
        compiler_params=pltpu.CompilerParams(use_tc_tiling_on_sc=False),
    )(_body)
    return run(x.astype(jnp.int32), text_table, pos_table)
